# baseline (speedup 1.0000x reference)
import functools

import jax
import jax.numpy as jnp
from jax import lax
from jax.experimental import pallas as pl
from jax.experimental.pallas import tpu as pltpu
from jax.experimental.pallas import tpu_sc as plsc

_B = 16384
_H1 = 4096
_H2 = _B - _H1
_DX = 128
_DE = 64
_INFO = plsc.get_sparse_core_info()
_NC = _INFO.num_cores
_NS = _INFO.num_subcores
_NW = _NC * _NS
_Q = 128
_CROWS = 2432
_BROWS = _CROWS // _NS

_mesh = plsc.VectorSubcoreMesh(core_axis_name="c", subcore_axis_name="s")


def _make_sc(start, nrows):
    build = start == 0
    _CHUNK = nrows // _NW
    _NQ = _CHUNK // _Q
    _GROUPS = _CHUNK // 16
    out_type = [jax.ShapeDtypeStruct((nrows, _DE), jnp.float32)]
    if build:
        out_type.append(
            jax.ShapeDtypeStruct((_NC * _CROWS, 2 * _DE), jnp.float32))
    scratch = [
        pltpu.VMEM((4, _CHUNK), jnp.int32),
        pltpu.VMEM((_NQ, _Q), jnp.int32),
        pltpu.VMEM((2, _Q, 2 * _DE), jnp.float32),
        pltpu.VMEM((2, _Q, _DE), jnp.float32),
        pltpu.SemaphoreType.DMA,
        pltpu.SemaphoreType.DMA,
    ]
    if build:
        scratch = [
            pltpu.VMEM((7, _DE), jnp.float32),
            pltpu.VMEM((7, _DE), jnp.float32),
            pltpu.VMEM((7, _DE), jnp.float32),
            pltpu.VMEM((7, _DE), jnp.float32),
            pltpu.VMEM((49, _DE), jnp.float32),
            pltpu.VMEM((49, _DE), jnp.float32),
            pltpu.VMEM((80, 2 * _DE), jnp.float32),
        ] + scratch

    def body(*refs):
        if build:
            (idx_hbm, t0_hbm, t1_hbm, t2_hbm, t3_hbm, emb_hbm, c_hbm,
             t0_v, t1_v, t2_v, t3_v, c01_v, c23_v, bld_v,
             idx_v, cidx_v, r_v, emb_v, gsem, osem) = refs
        else:
            (idx_hbm, c_hbm, emb_hbm,
             idx_v, cidx_v, r_v, emb_v, gsem, osem) = refs
        cid = lax.axis_index("c")
        sid = lax.axis_index("s")
        wid = sid * _NC + cid
        base = pl.multiple_of(wid * _CHUNK, _CHUNK)

        stage = [pltpu.async_copy(
            idx_hbm.at[:, pl.ds(start + base, _CHUNK)], idx_v, gsem)]
        if build:
            stage += [
                pltpu.async_copy(t0_hbm.at[pl.ds(0, 7), :], t0_v, gsem),
                pltpu.async_copy(t1_hbm.at[pl.ds(0, 7), :], t1_v, gsem),
                pltpu.async_copy(t2_hbm.at[pl.ds(0, 7), :], t2_v, gsem),
                pltpu.async_copy(t3_hbm.at[pl.ds(0, 7), :], t3_v, gsem),
            ]
        for c in stage:
            c.wait()

        if build:
            for i in range(7):
                for k in range(_DE // 16):
                    a0 = t0_v[i, pl.ds(16 * k, 16)]
                    a2 = t2_v[i, pl.ds(16 * k, 16)]
                    for j in range(7):
                        c01_v[7 * i + j, pl.ds(16 * k, 16)] = (
                            a0 + t1_v[j, pl.ds(16 * k, 16)])
                        c23_v[7 * i + j, pl.ds(16 * k, 16)] = (
                            a2 + t3_v[j, pl.ds(16 * k, 16)])
            for bstart, brows in ((0, 80), (80, 72)):
                hoff = _BROWS * sid + bstart

                def build_row(r, carry):
                    a = lax.rem(hoff + r, 49)
                    b = lax.div(hoff + r, 49)
                    bm = jnp.where(b > 48, 0, b)
                    for k in range(_DE // 16):
                        bld_v[r, pl.ds(16 * k, 16)] = (
                            c01_v[a, pl.ds(16 * k, 16)]
                            + c23_v[bm, pl.ds(16 * k, 16)])
                    return carry

                lax.fori_loop(0, brows, build_row, 0)
                coff = pl.multiple_of(_CROWS * cid + hoff, 8)
                pltpu.sync_copy(bld_v.at[pl.ds(0, brows), :],
                                c_hbm.at[pl.ds(coff, brows), :])

        tbase = _CROWS * cid
        for g in range(_GROUPS):
            q, m = divmod(g, _GROUPS // _NQ)
            i0 = idx_v[0, pl.ds(16 * g, 16)]
            i1 = idx_v[1, pl.ds(16 * g, 16)]
            i2 = idx_v[2, pl.ds(16 * g, 16)]
            i3 = idx_v[3, pl.ds(16 * g, 16)]
            cidx_v[q, pl.ds(16 * m, 16)] = (
                tbase + (7 * i0 + i1) + 49 * (7 * i2 + i3))

        if build:
            plsc.subcore_barrier()

        pending = []
        gath = [None, None]
        gstore = [None, None]
        gath[0] = pltpu.async_copy(c_hbm.at[cidx_v.at[0]], r_v.at[0], gsem)
        for p in range(_NQ):
            if p + 1 < _NQ:
                gath[(p + 1) % 2] = pltpu.async_copy(
                    c_hbm.at[cidx_v.at[p + 1]], r_v.at[(p + 1) % 2], gsem)
            if gstore[p % 2] is not None:
                gstore[p % 2].wait()
                pending = [c for c in pending if c is not gstore[p % 2]]
                gstore[p % 2] = None
            gath[p % 2].wait()

            def copy_rows(it, carry):
                for u in range(8):
                    row = 8 * it + u
                    for k in range(_DE // 16):
                        emb_v[p % 2, row, pl.ds(16 * k, 16)] = (
                            r_v[p % 2, row, pl.ds(16 * k, 16)])
                return carry

            lax.fori_loop(0, _Q // 8, copy_rows, 0)
            es = pltpu.async_copy(
                emb_v.at[p % 2],
                emb_hbm.at[pl.ds(base + _Q * p, _Q), :], osem)
            pending.append(es)
            gstore[p % 2] = es
        for c in pending:
            c.wait()

    return functools.partial(
        pl.kernel, mesh=_mesh, out_type=tuple(out_type),
        scratch_types=scratch)(body)


_sc0 = _make_sc(0, _H1)
_sc1 = _make_sc(_H1, _H2)

_BLK = 2048


def _tc_body1(x_ref, emb_ref, out_ref):
    out_ref[_DX:, :] = emb_ref[...].T
    out_ref[:_DX, :] = x_ref[...].T


def _tc_body2(x_ref, emb_ref, o_ref, out_ref):
    del o_ref
    out_ref[_DX:, :] = emb_ref[...].T
    out_ref[:_DX, :] = x_ref[...].T


@jax.jit
def kernel(x, x_time_encode, T0, T1, T2, T3):
    idx_t = x_time_encode.T
    emb1, c = _sc0(idx_t, T0, T1, T2, T3)
    (emb2,) = _sc1(idx_t, c)
    g1 = _H1 // _BLK
    g2 = _H2 // _BLK
    o1 = pl.pallas_call(
        _tc_body1,
        grid=(g1,),
        in_specs=[
            pl.BlockSpec((_BLK, _DX), lambda i: (i, 0)),
            pl.BlockSpec((_BLK, _DE), lambda i: (i, 0)),
        ],
        out_specs=pl.BlockSpec((_DX + _DE, _BLK), lambda i: (0, i)),
        out_shape=jax.ShapeDtypeStruct((_DX + _DE, _B), jnp.float32),
    )(x, emb1)
    out_t = pl.pallas_call(
        _tc_body2,
        grid=(g2,),
        in_specs=[
            pl.BlockSpec((_BLK, _DX), lambda i: (i + g1, 0)),
            pl.BlockSpec((_BLK, _DE), lambda i: (i, 0)),
            pl.BlockSpec(memory_space=pltpu.MemorySpace.HBM),
        ],
        out_specs=pl.BlockSpec((_DX + _DE, _BLK), lambda i: (0, i + g1)),
        out_shape=jax.ShapeDtypeStruct((_DX + _DE, _B), jnp.float32),
        input_output_aliases={2: 0},
    )(x, emb2, o1)
    return out_t.T


# speedup vs baseline: 1.0231x; 1.0231x over previous
import functools

import jax
import jax.numpy as jnp
from jax import lax
from jax.experimental import pallas as pl
from jax.experimental.pallas import tpu as pltpu
from jax.experimental.pallas import tpu_sc as plsc

_B = 16384
_H = _B // 2
_DX = 128
_DE = 64
_INFO = plsc.get_sparse_core_info()
_NC = _INFO.num_cores
_NS = _INFO.num_subcores
_NW = _NC * _NS
_CHUNK = _H // _NW
_Q = 128
_NQ = _CHUNK // _Q
_GROUPS = _CHUNK // 16
_CROWS = 2432
_BROWS = _CROWS // _NS

_mesh = plsc.VectorSubcoreMesh(core_axis_name="c", subcore_axis_name="s")


def _make_sc(h):
    build = h == 0
    out_type = [jax.ShapeDtypeStruct((_H, _DE), jnp.float32)]
    if build:
        out_type.append(
            jax.ShapeDtypeStruct((_NC * _CROWS, 2 * _DE), jnp.float32))
    scratch = [
        pltpu.VMEM((4, _CHUNK), jnp.int32),
        pltpu.VMEM((_NQ, _Q), jnp.int32),
        pltpu.VMEM((2, _Q, 2 * _DE), jnp.float32),
        pltpu.VMEM((2, _Q, _DE), jnp.float32),
        pltpu.SemaphoreType.DMA,
        pltpu.SemaphoreType.DMA,
    ]
    if build:
        scratch = [
            pltpu.VMEM((7, _DE), jnp.float32),
            pltpu.VMEM((7, _DE), jnp.float32),
            pltpu.VMEM((7, _DE), jnp.float32),
            pltpu.VMEM((7, _DE), jnp.float32),
            pltpu.VMEM((49, _DE), jnp.float32),
            pltpu.VMEM((49, _DE), jnp.float32),
            pltpu.VMEM((80, 2 * _DE), jnp.float32),
        ] + scratch

    def body(*refs):
        if build:
            (idx_hbm, t0_hbm, t1_hbm, t2_hbm, t3_hbm, emb_hbm, c_hbm,
             t0_v, t1_v, t2_v, t3_v, c01_v, c23_v, bld_v,
             idx_v, cidx_v, r_v, emb_v, gsem, osem) = refs
        else:
            (idx_hbm, c_hbm, emb_hbm,
             idx_v, cidx_v, r_v, emb_v, gsem, osem) = refs
        cid = lax.axis_index("c")
        sid = lax.axis_index("s")
        wid = sid * _NC + cid
        base = pl.multiple_of(wid * _CHUNK, _CHUNK)

        stage = [pltpu.async_copy(
            idx_hbm.at[:, pl.ds(h * _H + base, _CHUNK)], idx_v, gsem)]
        if build:
            stage += [
                pltpu.async_copy(t0_hbm.at[pl.ds(0, 7), :], t0_v, gsem),
                pltpu.async_copy(t1_hbm.at[pl.ds(0, 7), :], t1_v, gsem),
                pltpu.async_copy(t2_hbm.at[pl.ds(0, 7), :], t2_v, gsem),
                pltpu.async_copy(t3_hbm.at[pl.ds(0, 7), :], t3_v, gsem),
            ]
        for c in stage:
            c.wait()

        if build:
            for i in range(7):
                for k in range(_DE // 16):
                    a0 = t0_v[i, pl.ds(16 * k, 16)]
                    a2 = t2_v[i, pl.ds(16 * k, 16)]
                    for j in range(7):
                        c01_v[7 * i + j, pl.ds(16 * k, 16)] = (
                            a0 + t1_v[j, pl.ds(16 * k, 16)])
                        c23_v[7 * i + j, pl.ds(16 * k, 16)] = (
                            a2 + t3_v[j, pl.ds(16 * k, 16)])
            for start, nrows in ((0, 80), (80, 72)):
                hoff = _BROWS * sid + start

                def build_row(r, carry):
                    a = lax.rem(hoff + r, 49)
                    b = lax.div(hoff + r, 49)
                    bm = jnp.where(b > 48, 0, b)
                    for k in range(_DE // 16):
                        bld_v[r, pl.ds(16 * k, 16)] = (
                            c01_v[a, pl.ds(16 * k, 16)]
                            + c23_v[bm, pl.ds(16 * k, 16)])
                    return carry

                lax.fori_loop(0, nrows, build_row, 0)
                coff = pl.multiple_of(_CROWS * cid + hoff, 8)
                pltpu.sync_copy(bld_v.at[pl.ds(0, nrows), :],
                                c_hbm.at[pl.ds(coff, nrows), :])

        tbase = _CROWS * cid
        for g in range(_GROUPS):
            q, m = divmod(g, _GROUPS // _NQ)
            i0 = idx_v[0, pl.ds(16 * g, 16)]
            i1 = idx_v[1, pl.ds(16 * g, 16)]
            i2 = idx_v[2, pl.ds(16 * g, 16)]
            i3 = idx_v[3, pl.ds(16 * g, 16)]
            cidx_v[q, pl.ds(16 * m, 16)] = (
                tbase + (7 * i0 + i1) + 49 * (7 * i2 + i3))

        if build:
            plsc.subcore_barrier()

        pending = []
        gath = [None, None]
        gstore = [None, None]
        gath[0] = pltpu.async_copy(c_hbm.at[cidx_v.at[0]], r_v.at[0], gsem)
        for p in range(_NQ):
            if p + 1 < _NQ:
                gath[(p + 1) % 2] = pltpu.async_copy(
                    c_hbm.at[cidx_v.at[p + 1]], r_v.at[(p + 1) % 2], gsem)
            if gstore[p % 2] is not None:
                gstore[p % 2].wait()
                pending = [c for c in pending if c is not gstore[p % 2]]
                gstore[p % 2] = None
            gath[p % 2].wait()

            def copy_rows(it, carry):
                for u in range(8):
                    row = 8 * it + u
                    for k in range(_DE // 16):
                        emb_v[p % 2, row, pl.ds(16 * k, 16)] = (
                            r_v[p % 2, row, pl.ds(16 * k, 16)])
                return carry

            lax.fori_loop(0, _Q // 8, copy_rows, 0)
            es = pltpu.async_copy(
                emb_v.at[p % 2],
                emb_hbm.at[pl.ds(base + _Q * p, _Q), :], osem)
            pending.append(es)
            gstore[p % 2] = es
        for c in pending:
            c.wait()

    return functools.partial(
        pl.kernel, mesh=_mesh, out_type=tuple(out_type),
        scratch_types=scratch)(body)


_sc0 = _make_sc(0)
_sc1 = _make_sc(1)

_BLK = 2048


def _tc_body1(x_ref, emb_ref, out_ref):
    out_ref[_DX:, :] = emb_ref[...].T
    out_ref[:_DX, :] = x_ref[...].T


def _tc_body2(x_ref, emb_ref, o_ref, out_ref):
    del o_ref
    out_ref[_DX:, :] = emb_ref[...].T
    out_ref[:_DX, :] = x_ref[...].T


@jax.jit
def kernel(x, x_time_encode, T0, T1, T2, T3):
    idx_t = x_time_encode.T
    emb1, c = _sc0(idx_t, T0, T1, T2, T3)
    (emb2,) = _sc1(idx_t, c)
    grid = _H // _BLK
    o1 = pl.pallas_call(
        _tc_body1,
        grid=(grid,),
        in_specs=[
            pl.BlockSpec((_BLK, _DX), lambda i: (i, 0)),
            pl.BlockSpec((_BLK, _DE), lambda i: (i, 0)),
        ],
        out_specs=pl.BlockSpec((_DX + _DE, _BLK), lambda i: (0, i)),
        out_shape=jax.ShapeDtypeStruct((_DX + _DE, _B), jnp.float32),
    )(x, emb1)
    out_t = pl.pallas_call(
        _tc_body2,
        grid=(grid,),
        in_specs=[
            pl.BlockSpec((_BLK, _DX), lambda i: (i + grid, 0)),
            pl.BlockSpec((_BLK, _DE), lambda i: (i, 0)),
            pl.BlockSpec(memory_space=pltpu.MemorySpace.HBM),
        ],
        out_specs=pl.BlockSpec((_DX + _DE, _BLK), lambda i: (0, i + grid)),
        out_shape=jax.ShapeDtypeStruct((_DX + _DE, _B), jnp.float32),
        input_output_aliases={2: 0},
    )(x, emb2, o1)
    return out_t.T
